# baseline (speedup 1.0000x reference)
import functools

import jax
import jax.numpy as jnp
from jax import lax
from jax.experimental import pallas as pl
from jax.experimental.pallas import tpu as pltpu
from jax.experimental.pallas import tpu_sc as plsc

NUM_USERS = 20000
NUM_ITEMS = 30000
NNODES = NUM_USERS + NUM_ITEMS
DIM = 64
HALF = DIM // 2
N_LAYERS = 3

NC = 2
NS = 16

CHUNK = 128
SUP = 2
SUPE = CHUNK * SUP
NBUF = 3

CPT = -(-NNODES // NS // 8) * 8
CPT_LAST = NNODES - (NS - 1) * CPT
ACC_ROWS = ((NNODES + NS * CHUNK - 1) // (NS * CHUNK)) * (NS * CHUNK)
ZROWS_PER_TILE = ACC_ROWS // NS


def _spmm_body(nsup_tot, interleaved, x_hbm, cols_hbm, rows2_hbm, vals_hbm,
               out_hbm, colbufs, rowbufs, valbufs, gbufs, acc, insems, gsems,
               ssems):
  c = lax.axis_index("c")
  s = lax.axis_index("s")

  nb_ = nsup_tot // NS
  rem = nsup_tot % NS
  nsup = nb_ + jnp.where(s < rem, 1, 0)
  sup_base = s * nb_ + jnp.minimum(s, rem)

  zeros16 = jnp.zeros((16,), jnp.float32)

  zb = gbufs[0]
  def _zero_gbuf(i, _):
    zb[i, 0:16] = zeros16
    zb[i, 16:32] = zeros16
    return 0
  lax.fori_loop(0, SUPE, _zero_gbuf, 0)
  zbase = s * ZROWS_PER_TILE
  nfull = ZROWS_PER_TILE // SUPE
  for k in range(nfull):
    pltpu.sync_copy(zb, acc.at[pl.ds(zbase + k * SUPE, SUPE)])
  remz = ZROWS_PER_TILE - nfull * SUPE
  if remz:
    pltpu.sync_copy(zb.at[pl.ds(0, remz)], acc.at[pl.ds(zbase + nfull * SUPE, remz)])
  plsc.subcore_barrier()

  idx_add = c if interleaved else c * NNODES

  def fire_in(g, b):
    base = pl.multiple_of((sup_base + g) * SUPE, SUPE)
    rbase = pl.multiple_of((sup_base + g) * SUP, SUP)
    pltpu.async_copy(cols_hbm.at[pl.ds(base, SUPE)], colbufs[b], insems[b])
    pltpu.async_copy(vals_hbm.at[pl.ds(base, SUPE)], valbufs[b], insems[b])
    pltpu.async_copy(rows2_hbm.at[pl.ds(rbase, SUP)], rowbufs[b], insems[b])

  def wait_in(g, b):
    base = pl.multiple_of((sup_base + g) * SUPE, SUPE)
    rbase = pl.multiple_of((sup_base + g) * SUP, SUP)
    pltpu.make_async_copy(cols_hbm.at[pl.ds(base, SUPE)], colbufs[b], insems[b]).wait()
    pltpu.make_async_copy(vals_hbm.at[pl.ds(base, SUPE)], valbufs[b], insems[b]).wait()
    pltpu.make_async_copy(rows2_hbm.at[pl.ds(rbase, SUP)], rowbufs[b], insems[b]).wait()

  def transform_cols(b):
    cb = colbufs[b]
    for i in range(SUPE // 16):
      sl = pl.ds(i * 16, 16)
      if interleaved:
        cb[sl] = cb[sl] * 2 + idx_add
      else:
        cb[sl] = cb[sl] + idx_add

  def fire_gather(b):
    for j in range(SUP):
      pltpu.async_copy(x_hbm.at[colbufs[b].at[pl.ds(j * CHUNK, CHUNK)]],
                       gbufs[b].at[pl.ds(j * CHUNK, CHUNK)], gsems[b])

  def wait_gather(b):
    for j in range(SUP):
      pltpu.make_async_copy(x_hbm.at[colbufs[b].at[pl.ds(j * CHUNK, CHUNK)]],
                            gbufs[b].at[pl.ds(j * CHUNK, CHUNK)], gsems[b]).wait()

  splat_idx = [jnp.full((16,), k, jnp.int32) for k in range(16)]

  def scale(b):
    gbuf, valbuf = gbufs[b], valbufs[b]
    def _scale(t, _):
      vv = valbuf[pl.ds(t * 16, 16)]
      ebase = t * 16
      for k in range(16):
        sv = vv[k]
        gbuf[ebase + k, 0:16] = gbuf[ebase + k, 0:16] * sv
        gbuf[ebase + k, 16:32] = gbuf[ebase + k, 16:32] * sv
      return 0
    lax.fori_loop(0, SUPE // 16, _scale, 0)

  def scatter(b):
    for j in range(SUP):
      pltpu.async_copy(gbufs[b].at[pl.ds(j * CHUNK, CHUNK)],
                       acc.at[rowbufs[b].at[j]], ssems[b], add=True)

  def wait_scatter(b):
    for j in range(SUP):
      pltpu.make_async_copy(gbufs[b].at[pl.ds(j * CHUNK, CHUNK)],
                            acc.at[rowbufs[b].at[j]], ssems[b]).wait()

  fire_in(0, 0)

  @pl.when(nsup > 1)
  def _pro1():
    fire_in(1, 1)

  wait_in(0, 0)
  transform_cols(0)
  fire_gather(0)

  def _super(g, _):
    b = lax.rem(g, NBUF)
    for bi in range(NBUF):
      @pl.when(b == bi)
      def _slot():
        s1 = (bi + 1) % NBUF
        s2 = (bi + 2) % NBUF

        @pl.when(g + 2 < nsup)
        def _pre():
          fire_in(g + 2, s2)

        wait_gather(bi)
        scale(bi)
        scatter(bi)

        @pl.when(g + 1 < nsup)
        def _next():
          wait_in(g + 1, s1)
          transform_cols(s1)

          @pl.when(g >= 2)
          def _ws():
            wait_scatter(s1)

          fire_gather(s1)
    return 0

  lax.fori_loop(0, nsup, _super, 0)

  for bi in range(NBUF):
    wait_scatter(bi)

  plsc.subcore_barrier()

  off = c * NNODES
  abase = pl.multiple_of(s * CPT, 8)
  obase = pl.multiple_of(off + s * CPT, 8)
  pltpu.sync_copy(acc.at[pl.ds(abase, CPT_LAST)],
                  out_hbm.at[pl.ds(obase, CPT_LAST)])

  @pl.when(s < NS - 1)
  def _tail():
    pltpu.sync_copy(acc.at[pl.ds(abase + CPT_LAST, CPT - CPT_LAST)],
                    out_hbm.at[pl.ds(obase + CPT_LAST, CPT - CPT_LAST)])


def _make_spmm(nsup_tot, interleaved):
  mesh = plsc.VectorSubcoreMesh(core_axis_name="c", subcore_axis_name="s")
  body = functools.partial(_spmm_body, nsup_tot, interleaved)

  def wrapped(x_hbm, cols_hbm, rows2_hbm, vals_hbm, out_hbm, *scr):
    n = NBUF
    body(x_hbm, cols_hbm, rows2_hbm, vals_hbm, out_hbm,
         list(scr[0:n]), list(scr[n:2 * n]), list(scr[2 * n:3 * n]),
         list(scr[3 * n:4 * n]), scr[4 * n],
         list(scr[4 * n + 1:5 * n + 1]), list(scr[5 * n + 1:6 * n + 1]),
         list(scr[6 * n + 1:7 * n + 1]))

  return pl.kernel(
      wrapped,
      out_type=jax.ShapeDtypeStruct((NC * NNODES, HALF), jnp.float32),
      mesh=mesh,
      compiler_params=pltpu.CompilerParams(use_tc_tiling_on_sc=False),
      scratch_types=(
          [pltpu.VMEM((SUPE,), jnp.int32)] * NBUF
          + [pltpu.VMEM((SUP, CHUNK), jnp.int32)] * NBUF
          + [pltpu.VMEM((SUPE,), jnp.float32)] * NBUF
          + [pltpu.VMEM((SUPE, HALF), jnp.float32)] * NBUF
          + [pltpu.VMEM_SHARED((ACC_ROWS, HALF), jnp.float32)]
          + [pltpu.SemaphoreType.DMA] * (3 * NBUF)
      ),
  )


def _mean_body(e_ref, a0, a1, a2, b0, b1, b2, out):
  ev = e_ref[...]
  lo = a0[...] + a1[...] + a2[...]
  hi = b0[...] + b1[...] + b2[...]
  out[:, 0:HALF] = (ev[:, 0:HALF] + lo) * 0.25
  out[:, HALF:DIM] = (ev[:, HALF:DIM] + hi) * 0.25


_MEAN_BR = 2000


def _mean_call(emb, x1, x2, x3):
  nblk = NNODES // _MEAN_BR
  e_spec = pl.BlockSpec((_MEAN_BR, DIM), lambda i: (i, 0))
  lo_spec = pl.BlockSpec((_MEAN_BR, HALF), lambda i: (i, 0))
  hi_spec = pl.BlockSpec((_MEAN_BR, HALF), lambda i, n=nblk: (i + n, 0))
  return pl.pallas_call(
      _mean_body,
      grid=(nblk,),
      in_specs=[e_spec] + [lo_spec] * 3 + [hi_spec] * 3,
      out_specs=pl.BlockSpec((_MEAN_BR, DIM), lambda i: (i, 0)),
      out_shape=jax.ShapeDtypeStruct((NNODES, DIM), jnp.float32),
  )(emb, x1, x2, x3, x1, x2, x3)


@jax.jit
def kernel(embeddings, adjacency_indices, adjacency_values):
  nnz = adjacency_values.shape[0]
  nnz_pad = -(-nnz // SUPE) * SUPE
  pad = nnz_pad - nnz
  nsup_tot = nnz_pad // SUPE

  rows = adjacency_indices[0]
  cols = adjacency_indices[1]
  if pad:
    rows = jnp.concatenate([rows, jnp.zeros((pad,), jnp.int32)])
    cols = jnp.concatenate([cols, jnp.zeros((pad,), jnp.int32)])
    vals = jnp.concatenate([adjacency_values, jnp.zeros((pad,), jnp.float32)])
  else:
    vals = adjacency_values
  rows2 = rows.reshape(nnz_pad // CHUNK, CHUNK)

  x0 = embeddings.reshape(NC * NNODES, HALF)
  x1 = _make_spmm(nsup_tot, True)(x0, cols, rows2, vals)
  spmm = _make_spmm(nsup_tot, False)
  x2 = spmm(x1, cols, rows2, vals)
  x3 = spmm(x2, cols, rows2, vals)

  out = _mean_call(embeddings, x1, x2, x3)
  return out[:NUM_USERS], out[NUM_USERS:NNODES]


# speedup vs baseline: 1.4793x; 1.4793x over previous
import functools

import jax
import jax.numpy as jnp
from jax import lax
from jax.experimental import pallas as pl
from jax.experimental.pallas import tpu as pltpu
from jax.experimental.pallas import tpu_sc as plsc

NUM_USERS = 20000
NUM_ITEMS = 30000
NNODES = NUM_USERS + NUM_ITEMS
DIM = 64
HALF = DIM // 2
N_LAYERS = 3

NC = 2
NS = 16

CHUNK = 128
SUP = 2
SUPE = CHUNK * SUP
NBUF = 3

CPT = -(-NNODES // NS // 8) * 8
CPT_LAST = NNODES - (NS - 1) * CPT
ACC_ROWS = ((NNODES + NS * CHUNK - 1) // (NS * CHUNK)) * (NS * CHUNK)
ZROWS_PER_TILE = ACC_ROWS // NS


def _spmm_body(nsup_tot, interleaved, x_hbm, cols_hbm, rows2_hbm, vals_hbm,
               out_hbm, colbufs, rowbufs, valbufs, gbufs, acc, insems, gsems,
               ssems):
  c = lax.axis_index("c")
  s = lax.axis_index("s")

  nb_ = nsup_tot // NS
  rem = nsup_tot % NS
  nsup = nb_ + jnp.where(s < rem, 1, 0)
  sup_base = s * nb_ + jnp.minimum(s, rem)

  zeros16 = jnp.zeros((16,), jnp.float32)

  zb = gbufs[0]
  def _zero_gbuf(i, _):
    zb[i, 0:16] = zeros16
    zb[i, 16:32] = zeros16
    return 0
  lax.fori_loop(0, SUPE, _zero_gbuf, 0)
  zbase = s * ZROWS_PER_TILE
  nfull = ZROWS_PER_TILE // SUPE
  for k in range(nfull):
    pltpu.sync_copy(zb, acc.at[pl.ds(zbase + k * SUPE, SUPE)])
  remz = ZROWS_PER_TILE - nfull * SUPE
  if remz:
    pltpu.sync_copy(zb.at[pl.ds(0, remz)], acc.at[pl.ds(zbase + nfull * SUPE, remz)])
  plsc.subcore_barrier()

  idx_add = c if interleaved else c * NNODES

  def fire_in(g, b):
    base = pl.multiple_of((sup_base + g) * SUPE, SUPE)
    rbase = pl.multiple_of((sup_base + g) * SUP, SUP)
    pltpu.async_copy(cols_hbm.at[pl.ds(base, SUPE)], colbufs[b], insems[b])
    pltpu.async_copy(vals_hbm.at[pl.ds(base, SUPE)], valbufs[b], insems[b])
    pltpu.async_copy(rows2_hbm.at[pl.ds(rbase, SUP)], rowbufs[b], insems[b])

  def wait_in(g, b):
    base = pl.multiple_of((sup_base + g) * SUPE, SUPE)
    rbase = pl.multiple_of((sup_base + g) * SUP, SUP)
    pltpu.make_async_copy(cols_hbm.at[pl.ds(base, SUPE)], colbufs[b], insems[b]).wait()
    pltpu.make_async_copy(vals_hbm.at[pl.ds(base, SUPE)], valbufs[b], insems[b]).wait()
    pltpu.make_async_copy(rows2_hbm.at[pl.ds(rbase, SUP)], rowbufs[b], insems[b]).wait()

  def transform_cols(b):
    cb = colbufs[b]
    for i in range(SUPE // 16):
      sl = pl.ds(i * 16, 16)
      if interleaved:
        cb[sl] = cb[sl] * 2 + idx_add
      else:
        cb[sl] = cb[sl] + idx_add

  def fire_gather(b):
    for j in range(SUP):
      pltpu.async_copy(x_hbm.at[colbufs[b].at[pl.ds(j * CHUNK, CHUNK)]],
                       gbufs[b].at[pl.ds(j * CHUNK, CHUNK)], gsems[b])

  def wait_gather(b):
    for j in range(SUP):
      pltpu.make_async_copy(x_hbm.at[colbufs[b].at[pl.ds(j * CHUNK, CHUNK)]],
                            gbufs[b].at[pl.ds(j * CHUNK, CHUNK)], gsems[b]).wait()

  splat_idx = [jnp.full((16,), k, jnp.int32) for k in range(16)]

  def scale(b):
    gbuf, valbuf = gbufs[b], valbufs[b]
    def _scale(t, _):
      vv = valbuf[pl.ds(t * 16, 16)]
      ebase = t * 16
      for k in range(16):
        sv = vv[k]
        gbuf[ebase + k, 0:16] = gbuf[ebase + k, 0:16] * sv
        gbuf[ebase + k, 16:32] = gbuf[ebase + k, 16:32] * sv
      return 0
    lax.fori_loop(0, SUPE // 16, _scale, 0)

  def scatter(b):
    for j in range(SUP):
      pltpu.async_copy(gbufs[b].at[pl.ds(j * CHUNK, CHUNK)],
                       acc.at[rowbufs[b].at[j]], ssems[b], add=True)

  def wait_scatter(b):
    for j in range(SUP):
      pltpu.make_async_copy(gbufs[b].at[pl.ds(j * CHUNK, CHUNK)],
                            acc.at[rowbufs[b].at[j]], ssems[b]).wait()

  fire_in(0, 0)

  @pl.when(nsup > 1)
  def _pro1():
    fire_in(1, 1)

  wait_in(0, 0)
  transform_cols(0)
  fire_gather(0)

  def _super(g, _):
    b = lax.rem(g, NBUF)
    for bi in range(NBUF):
      @pl.when(b == bi)
      def _slot():
        s1 = (bi + 1) % NBUF
        s2 = (bi + 2) % NBUF

        @pl.when(g + 2 < nsup)
        def _pre():
          fire_in(g + 2, s2)

        @pl.when(g + 1 < nsup)
        def _next():
          wait_in(g + 1, s1)
          transform_cols(s1)

          @pl.when(g >= 2)
          def _ws():
            wait_scatter(s1)

          fire_gather(s1)

        wait_gather(bi)
        scale(bi)
        scatter(bi)
    return 0

  lax.fori_loop(0, nsup, _super, 0)

  for bi in range(NBUF):
    wait_scatter(bi)

  plsc.subcore_barrier()

  off = c * NNODES
  abase = pl.multiple_of(s * CPT, 8)
  obase = pl.multiple_of(off + s * CPT, 8)
  pltpu.sync_copy(acc.at[pl.ds(abase, CPT_LAST)],
                  out_hbm.at[pl.ds(obase, CPT_LAST)])

  @pl.when(s < NS - 1)
  def _tail():
    pltpu.sync_copy(acc.at[pl.ds(abase + CPT_LAST, CPT - CPT_LAST)],
                    out_hbm.at[pl.ds(obase + CPT_LAST, CPT - CPT_LAST)])


def _make_spmm(nsup_tot, interleaved):
  mesh = plsc.VectorSubcoreMesh(core_axis_name="c", subcore_axis_name="s")
  body = functools.partial(_spmm_body, nsup_tot, interleaved)

  def wrapped(x_hbm, cols_hbm, rows2_hbm, vals_hbm, out_hbm, *scr):
    n = NBUF
    body(x_hbm, cols_hbm, rows2_hbm, vals_hbm, out_hbm,
         list(scr[0:n]), list(scr[n:2 * n]), list(scr[2 * n:3 * n]),
         list(scr[3 * n:4 * n]), scr[4 * n],
         list(scr[4 * n + 1:5 * n + 1]), list(scr[5 * n + 1:6 * n + 1]),
         list(scr[6 * n + 1:7 * n + 1]))

  return pl.kernel(
      wrapped,
      out_type=jax.ShapeDtypeStruct((NC * NNODES, HALF), jnp.float32),
      mesh=mesh,
      compiler_params=pltpu.CompilerParams(use_tc_tiling_on_sc=False),
      scratch_types=(
          [pltpu.VMEM((SUPE,), jnp.int32)] * NBUF
          + [pltpu.VMEM((SUP, CHUNK), jnp.int32)] * NBUF
          + [pltpu.VMEM((SUPE,), jnp.float32)] * NBUF
          + [pltpu.VMEM((SUPE, HALF), jnp.float32)] * NBUF
          + [pltpu.VMEM_SHARED((ACC_ROWS, HALF), jnp.float32)]
          + [pltpu.SemaphoreType.DMA] * (3 * NBUF)
      ),
  )


def _mean_body(e_ref, a0, a1, a2, b0, b1, b2, out):
  ev = e_ref[...]
  lo = a0[...] + a1[...] + a2[...]
  hi = b0[...] + b1[...] + b2[...]
  out[:, 0:HALF] = (ev[:, 0:HALF] + lo) * 0.25
  out[:, HALF:DIM] = (ev[:, HALF:DIM] + hi) * 0.25


_MEAN_BR = 2000


def _mean_call(emb, x1, x2, x3):
  nblk = NNODES // _MEAN_BR
  e_spec = pl.BlockSpec((_MEAN_BR, DIM), lambda i: (i, 0))
  lo_spec = pl.BlockSpec((_MEAN_BR, HALF), lambda i: (i, 0))
  hi_spec = pl.BlockSpec((_MEAN_BR, HALF), lambda i, n=nblk: (i + n, 0))
  return pl.pallas_call(
      _mean_body,
      grid=(nblk,),
      in_specs=[e_spec] + [lo_spec] * 3 + [hi_spec] * 3,
      out_specs=pl.BlockSpec((_MEAN_BR, DIM), lambda i: (i, 0)),
      out_shape=jax.ShapeDtypeStruct((NNODES, DIM), jnp.float32),
  )(emb, x1, x2, x3, x1, x2, x3)


@jax.jit
def kernel(embeddings, adjacency_indices, adjacency_values):
  nnz = adjacency_values.shape[0]
  nnz_pad = -(-nnz // SUPE) * SUPE
  pad = nnz_pad - nnz
  nsup_tot = nnz_pad // SUPE

  rows = adjacency_indices[0]
  cols = adjacency_indices[1]
  if pad:
    rows = jnp.concatenate([rows, jnp.zeros((pad,), jnp.int32)])
    cols = jnp.concatenate([cols, jnp.zeros((pad,), jnp.int32)])
    vals = jnp.concatenate([adjacency_values, jnp.zeros((pad,), jnp.float32)])
  else:
    vals = adjacency_values
  rows2 = rows.reshape(nnz_pad // CHUNK, CHUNK)

  x0 = embeddings.reshape(NC * NNODES, HALF)
  x1 = _make_spmm(nsup_tot, True)(x0, cols, rows2, vals)
  spmm = _make_spmm(nsup_tot, False)
  x2 = spmm(x1, cols, rows2, vals)
  x3 = spmm(x2, cols, rows2, vals)

  out = _mean_call(embeddings, x1, x2, x3)
  return out[:NUM_USERS], out[NUM_USERS:NNODES]
